# baseline (speedup 1.0000x reference)
import functools

import jax
import jax.numpy as jnp
from jax import lax
from jax.experimental import pallas as pl
from jax.experimental.pallas import tpu as pltpu
from jax.experimental.pallas import tpu_sc as plsc

NV = 64
NPIX = 512
PIXSCALE = 0.5
VEL0 = 0.0
DV = 10.0
FOV_HALF = 0.5 * (NPIX - 1) * PIXSCALE

NC, NS, L = 2, 16, 16
NW = NC * NS
M = 250000 * 8
CP = 2048
ITERS = CP // L
NCHUNK = 31
P_TILE = CP * NCHUNK
M_PAD = P_TILE * NW
NB = 256
NREC = M_PAD
RECW = 16
CR = 1024
ACC = 129 * 512

_mesh = plsc.VectorSubcoreMesh(core_axis_name="c", subcore_axis_name="s",
                               num_cores=NC, num_subcores=NS)
_sc_params = pltpu.CompilerParams(needs_layout_passes=False,
                                  use_tc_tiling_on_sc=False)


def _point_math(r, d, v, f):
    x = r * (1.0 / PIXSCALE) + (FOV_HALF / PIXSCALE)
    y = d * (1.0 / PIXSCALE) + (FOV_HALF / PIXSCALE)
    vz = v * (1.0 / DV) + (-VEL0 / DV)
    ok = ((x >= 0.0) & (x < NPIX - 1.0) & (y >= 0.0) & (y < NPIX - 1.0)
          & (vz >= 0.0) & (vz < NV - 1.0))
    f = jnp.where(ok, f, 0.0)
    ix = jnp.clip(x.astype(jnp.int32), 0, NPIX - 2)
    iy = jnp.clip(y.astype(jnp.int32), 0, NPIX - 2)
    iv = jnp.clip(vz.astype(jnp.int32), 0, NV - 2)
    fx = jnp.clip(x - ix.astype(jnp.float32), 0.0, 1.0)
    fy = jnp.clip(y - iy.astype(jnp.float32), 0.0, 1.0)
    fv = jnp.clip(vz - iv.astype(jnp.float32), 0.0, 1.0)
    yq = iy >> 7
    ly = iy & 127
    idx = ly * NPIX + ix
    b = iv * 4 + yq
    return b, idx, fx, fy, fv, f


@functools.partial(
    pl.kernel,
    out_type=jax.ShapeDtypeStruct((NB, NW, L), jnp.int32),
    mesh=_mesh,
    compiler_params=_sc_params,
    scratch_types=[
        pltpu.VMEM((CP,), jnp.float32),
        pltpu.VMEM((CP,), jnp.float32),
        pltpu.VMEM((CP,), jnp.float32),
        pltpu.VMEM((CP,), jnp.float32),
        pltpu.VMEM((NB, L), jnp.int32),
        pltpu.SemaphoreType.DMA,
    ],
)
def _p1a(ra, dec, vel, flx, counts, rb, db, vb, fb, cnt, sem):
    w = lax.axis_index("c") * NS + lax.axis_index("s")
    lane = lax.iota(jnp.int32, L)
    ones = jnp.ones((L,), jnp.int32)

    def zero_body(i, _):
        plsc.store_scatter(cnt, [jnp.full((L,), i, jnp.int32), lane],
                           jnp.zeros((L,), jnp.int32))
        return 0
    lax.fori_loop(0, NB, zero_body, 0)

    def chunk_body(ch, _):
        base = w * P_TILE + ch * CP
        cps = [pltpu.async_copy(src.at[pl.ds(base, CP)], dst, sem)
               for src, dst in ((ra, rb), (dec, db), (vel, vb), (flx, fb))]
        for cp in cps:
            cp.wait()

        def it_body(i, _):
            sl = pl.ds(i * L, L)
            b, _idx, _fx, _fy, _fv, _f = _point_math(rb[sl], db[sl], vb[sl], fb[sl])
            plsc.addupdate_scatter(cnt, [b, lane], ones)
            return 0
        lax.fori_loop(0, ITERS, it_body, 0)
        return 0
    lax.fori_loop(0, NCHUNK, chunk_body, 0)
    pltpu.sync_copy(cnt, counts.at[:, w, :])


_SCAN_CB = 16


@functools.partial(
    pl.kernel,
    out_type=(
        jax.ShapeDtypeStruct((NB, NW, L), jnp.int32),
        jax.ShapeDtypeStruct((2 * NB,), jnp.int32),
    ),
    mesh=_mesh,
    compiler_params=_sc_params,
    scratch_types=[
        pltpu.VMEM((_SCAN_CB, NW, L), jnp.int32),
        pltpu.VMEM((_SCAN_CB, NW, L), jnp.int32),
        pltpu.VMEM((2 * NB,), jnp.int32),
        pltpu.SemaphoreType.DMA,
    ],
)
def _p1b(counts, offsets, binfo, cbuf, obuf, bbuf, sem):
    w = lax.axis_index("c") * NS + lax.axis_index("s")

    @pl.when(w == 0)
    def _():
        lane = lax.iota(jnp.int32, L)

        def chunk_body(cb, carry):
            pltpu.sync_copy(counts.at[pl.ds(cb * _SCAN_CB, _SCAN_CB)], cbuf)

            def bucket_body(bl, carry):
                bglob = cb * _SCAN_CB + bl
                blv = jnp.full((L,), bl, jnp.int32)

                def vec_body(j, run):
                    jv = jnp.full((L,), j, jnp.int32)
                    v = plsc.load_gather(cbuf, [blv, jv, lane])
                    cs = plsc.cumsum(v)
                    off = (carry + run) + cs - v
                    plsc.store_scatter(obuf, [blv, jv, lane], off)
                    return run + cs[L - 1]
                run = lax.fori_loop(0, NW, vec_body, jnp.int32(0))
                bgv = jnp.full((L,), bglob, jnp.int32)
                m0 = lane == 0
                plsc.store_scatter(bbuf, [bgv], jnp.full((L,), carry, jnp.int32),
                                   mask=m0)
                plsc.store_scatter(bbuf, [bgv + NB], jnp.full((L,), run, jnp.int32),
                                   mask=m0)
                return carry + run
            carry = lax.fori_loop(0, _SCAN_CB, bucket_body, carry)
            pltpu.sync_copy(obuf, offsets.at[pl.ds(cb * _SCAN_CB, _SCAN_CB)])
            return carry
        lax.fori_loop(0, NB // _SCAN_CB, chunk_body, jnp.int32(0))
        pltpu.sync_copy(bbuf, binfo)


@functools.partial(
    pl.kernel,
    out_type=jax.ShapeDtypeStruct((NREC + CR, RECW), jnp.float32),
    mesh=_mesh,
    compiler_params=_sc_params,
    scratch_types=[
        pltpu.VMEM((CP,), jnp.float32),
        pltpu.VMEM((CP,), jnp.float32),
        pltpu.VMEM((CP,), jnp.float32),
        pltpu.VMEM((CP,), jnp.float32),
        pltpu.VMEM((NB, L), jnp.int32),
        pltpu.VMEM((2, ITERS * L, RECW), jnp.float32),
        pltpu.VMEM((2, ITERS * L // 128, 128), jnp.int32),
        pltpu.SemaphoreType.DMA,
        pltpu.SemaphoreType.DMA,
    ],
)
def _p1c(ra, dec, vel, flx, offsets, rec, rb, db, vb, fb, cur, stage, slots,
         sem_in, sem_out):
    w = lax.axis_index("c") * NS + lax.axis_index("s")
    lane = lax.iota(jnp.int32, L)
    pltpu.sync_copy(offsets.at[:, w, :], cur)
    ndma = ITERS * L // 128

    def chunk_body(ch, _):
        base = w * P_TILE + ch * CP
        cps = [pltpu.async_copy(src.at[pl.ds(base, CP)], dst, sem_in)
               for src, dst in ((ra, rb), (dec, db), (vel, vb), (flx, fb))]
        for cp in cps:
            cp.wait()
        pg = ch & 1

        def it_body(i, _):
            sl = pl.ds(i * L, L)
            b, idx, fx, fy, fv, f = _point_math(rb[sl], db[sl], vb[sl], fb[sl])
            slot = plsc.load_gather(cur, [b, lane])
            plsc.store_scatter(cur, [b, lane], slot + 1)
            row = jnp.full((L,), i * L, jnp.int32) + lane
            idxf = plsc.bitcast(idx, jnp.float32)
            pgv = jnp.full((L,), pg, jnp.int32)
            for col, valv in ((0, idxf), (1, fx), (2, fy),
                              (3, f * (1.0 - fv)), (4, f * fv)):
                plsc.store_scatter(stage, [pgv, row, jnp.full((L,), col, jnp.int32)],
                                   valv)
            plsc.store_scatter(slots,
                               [pgv, jnp.full((L,), i >> 3, jnp.int32),
                                jnp.full((L,), (i & 7) * L, jnp.int32) + lane],
                               slot)
            return 0
        lax.fori_loop(0, ITERS, it_body, 0)

        def fire(j, _):
            pltpu.async_copy(stage.at[pg, pl.ds(j * 128, 128)],
                             rec.at[slots.at[pg, j]], sem_out)
            return 0
        lax.fori_loop(0, ndma, fire, 0)

        @pl.when(ch >= 1)
        def _():
            og = 1 - pg
            pltpu.make_async_copy(rec.at[pl.ds(0, ITERS * L)], stage.at[og],
                                  sem_out).wait()
        return 0
    lax.fori_loop(0, NCHUNK, chunk_body, 0)
    pltpu.make_async_copy(rec.at[pl.ds(0, ITERS * L)],
                          stage.at[(NCHUNK - 1) & 1], sem_out).wait()


@functools.partial(
    pl.kernel,
    out_type=(
        jax.ShapeDtypeStruct((NV * NPIX * NPIX,), jnp.float32),
        jax.ShapeDtypeStruct((NB, NPIX), jnp.float32),
    ),
    mesh=_mesh,
    compiler_params=_sc_params,
    scratch_types=[
        pltpu.VMEM((ACC,), jnp.float32),
        pltpu.VMEM((2, CR, RECW), jnp.float32),
        pltpu.VMEM((2 * NB,), jnp.int32),
        pltpu.SemaphoreType.DMA,
    ],
)
def _p2(rec, binfo, cube, halos, acc, buf, bi, sem):
    w = lax.axis_index("c") * NS + lax.axis_index("s")
    lane = lax.iota(jnp.int32, L)
    pltpu.sync_copy(binfo, bi)
    zeros = jnp.zeros((L,), jnp.float32)

    def block_body(blk, _):
        bid = w * 8 + blk
        c = bid >> 2
        q = bid & 3

        def zb(i, _):
            acc[pl.ds(i * L, L)] = zeros
            return 0
        lax.fori_loop(0, ACC // L, zb, 0)

        def do_half(bucket, valcol, enable):
            bkc = jnp.clip(bucket, 0, NB - 1)
            start = plsc.load_gather(bi, [jnp.full((L,), bkc, jnp.int32)])[0]
            n = plsc.load_gather(bi, [jnp.full((L,), bkc + NB, jnp.int32)])[0]
            n = jnp.where(enable, n, 0)
            nch = (n + (CR - 1)) >> 10

            def ch_body(ch, _):
                pg = ch & 1
                pltpu.sync_copy(rec.at[pl.ds(start + ch * CR, CR)],
                                buf.at[pg])
                rem0 = n - ch * CR

                def grp(g, _):
                    row = jnp.full((L,), g * L, jnp.int32) + lane
                    gv = lambda col: plsc.load_gather(
                        buf, [jnp.full((L,), pg, jnp.int32), row,
                              jnp.full((L,), col, jnp.int32)])
                    idx = plsc.bitcast(gv(0), jnp.int32)
                    fx = gv(1)
                    fy = gv(2)
                    val = gv(valcol)
                    m = row < rem0
                    wy0 = val * (1.0 - fy)
                    wy1 = val * fy
                    plsc.addupdate_scatter(acc, [idx], wy0 * (1.0 - fx), mask=m)
                    plsc.addupdate_scatter(acc, [idx + 1], wy0 * fx, mask=m)
                    plsc.addupdate_scatter(acc, [idx + NPIX], wy1 * (1.0 - fx),
                                           mask=m)
                    plsc.addupdate_scatter(acc, [idx + NPIX + 1], wy1 * fx,
                                           mask=m)
                    return 0
                lax.fori_loop(0, CR // L, grp, 0)
                return 0
            lax.fori_loop(0, nch, ch_body, 0)

        do_half(c * 4 + q, 3, c <= NV - 2)
        do_half((c - 1) * 4 + q, 4, c >= 1)
        pltpu.sync_copy(acc.at[pl.ds(0, 128 * NPIX)],
                        cube.at[pl.ds((c * NPIX + q * 128) * NPIX, 128 * NPIX)])
        pltpu.sync_copy(acc.at[pl.ds(128 * NPIX, NPIX)], halos.at[bid])
        return 0
    lax.fori_loop(0, 8, block_body, 0)


def _p3_body(c_ref, h_ref, o_ref):
    o_ref[...] = c_ref[...]
    o_ref[0, 128, :] = c_ref[0, 128, :] + h_ref[0, 0, :]
    o_ref[0, 256, :] = c_ref[0, 256, :] + h_ref[0, 1, :]
    o_ref[0, 384, :] = c_ref[0, 384, :] + h_ref[0, 2, :]


def _p3(cube_raw, halos):
    return pl.pallas_call(
        _p3_body,
        grid=(NV,),
        in_specs=[
            pl.BlockSpec((1, NPIX, NPIX), lambda i: (i, 0, 0)),
            pl.BlockSpec((1, 4, NPIX), lambda i: (i, 0, 0)),
        ],
        out_specs=pl.BlockSpec((1, NPIX, NPIX), lambda i: (i, 0, 0)),
        out_shape=jax.ShapeDtypeStruct((NV, NPIX, NPIX), jnp.float32),
    )(cube_raw, halos)


def kernel(pos_img, vel_chan, flux):
    ra = pos_img[..., 0].reshape(M)
    dec = pos_img[..., 1].reshape(M)
    v = vel_chan.reshape(M)
    f = flux.reshape(M)
    pad = M_PAD - M
    ra = jnp.concatenate([ra, jnp.zeros((pad,), jnp.float32)])
    dec = jnp.concatenate([dec, jnp.zeros((pad,), jnp.float32)])
    v = jnp.concatenate([v, jnp.zeros((pad,), jnp.float32)])
    f = jnp.concatenate([f, jnp.zeros((pad,), jnp.float32)])

    counts = _p1a(ra, dec, v, f)
    offsets, binfo = _p1b(counts)
    rec = _p1c(ra, dec, v, f, offsets)
    cube_raw, halos = _p2(rec, binfo)
    cube = _p3(cube_raw.reshape(NV, NPIX, NPIX), halos.reshape(NV, 4, NPIX))
    return cube


# speedup vs baseline: 1.1926x; 1.1926x over previous
import functools

import jax
import jax.numpy as jnp
from jax import lax
from jax.experimental import pallas as pl
from jax.experimental.pallas import tpu as pltpu
from jax.experimental.pallas import tpu_sc as plsc

NV = 64
NPIX = 512
PIXSCALE = 0.5
VEL0 = 0.0
DV = 10.0
FOV_HALF = 0.5 * (NPIX - 1) * PIXSCALE

NC, NS, L = 2, 16, 16
NW = NC * NS
M = 250000 * 8
CP = 2048
ITERS = CP // L
NCHUNK = 31
P_TILE = CP * NCHUNK
M_PAD = P_TILE * NW
NB = 256
NREC = M_PAD
RECW = 16
CR = 1024
ACC = 129 * 512

_mesh = plsc.VectorSubcoreMesh(core_axis_name="c", subcore_axis_name="s",
                               num_cores=NC, num_subcores=NS)
_sc_params = pltpu.CompilerParams(needs_layout_passes=False,
                                  use_tc_tiling_on_sc=False)


def _point_math(r, d, v, f):
    x = r * (1.0 / PIXSCALE) + (FOV_HALF / PIXSCALE)
    y = d * (1.0 / PIXSCALE) + (FOV_HALF / PIXSCALE)
    vz = v * (1.0 / DV) + (-VEL0 / DV)
    ok = ((x >= 0.0) & (x < NPIX - 1.0) & (y >= 0.0) & (y < NPIX - 1.0)
          & (vz >= 0.0) & (vz < NV - 1.0))
    f = jnp.where(ok, f, 0.0)
    ix = jnp.clip(x.astype(jnp.int32), 0, NPIX - 2)
    iy = jnp.clip(y.astype(jnp.int32), 0, NPIX - 2)
    iv = jnp.clip(vz.astype(jnp.int32), 0, NV - 2)
    fx = jnp.clip(x - ix.astype(jnp.float32), 0.0, 1.0)
    fy = jnp.clip(y - iy.astype(jnp.float32), 0.0, 1.0)
    fv = jnp.clip(vz - iv.astype(jnp.float32), 0.0, 1.0)
    yq = iy >> 7
    ly = iy & 127
    idx = ly * NPIX + ix
    b = iv * 4 + yq
    return b, idx, fx, fy, fv, f


@functools.partial(
    pl.kernel,
    out_type=jax.ShapeDtypeStruct((NB, NW, L), jnp.int32),
    mesh=_mesh,
    compiler_params=_sc_params,
    scratch_types=[
        pltpu.VMEM((CP,), jnp.float32),
        pltpu.VMEM((CP,), jnp.float32),
        pltpu.VMEM((CP,), jnp.float32),
        pltpu.VMEM((CP,), jnp.float32),
        pltpu.VMEM((NB, L), jnp.int32),
        pltpu.SemaphoreType.DMA,
    ],
)
def _p1a(ra, dec, vel, flx, counts, rb, db, vb, fb, cnt, sem):
    w = lax.axis_index("c") * NS + lax.axis_index("s")
    lane = lax.iota(jnp.int32, L)
    ones = jnp.ones((L,), jnp.int32)

    def zero_body(i, _):
        plsc.store_scatter(cnt, [jnp.full((L,), i, jnp.int32), lane],
                           jnp.zeros((L,), jnp.int32))
        return 0
    lax.fori_loop(0, NB, zero_body, 0)

    def chunk_body(ch, _):
        base = w * P_TILE + ch * CP
        cps = [pltpu.async_copy(src.at[pl.ds(base, CP)], dst, sem)
               for src, dst in ((ra, rb), (dec, db), (vel, vb), (flx, fb))]
        for cp in cps:
            cp.wait()

        def it_body(i, _):
            sl = pl.ds(i * L, L)
            b, _idx, _fx, _fy, _fv, _f = _point_math(rb[sl], db[sl], vb[sl], fb[sl])
            plsc.addupdate_scatter(cnt, [b, lane], ones)
            return 0
        lax.fori_loop(0, ITERS, it_body, 0)
        return 0
    lax.fori_loop(0, NCHUNK, chunk_body, 0)
    pltpu.sync_copy(cnt, counts.at[:, w, :])


_SCAN_CB = 16


@functools.partial(
    pl.kernel,
    out_type=(
        jax.ShapeDtypeStruct((NB, NW, L), jnp.int32),
        jax.ShapeDtypeStruct((2 * NB,), jnp.int32),
    ),
    mesh=_mesh,
    compiler_params=_sc_params,
    scratch_types=[
        pltpu.VMEM((_SCAN_CB, NW, L), jnp.int32),
        pltpu.VMEM((_SCAN_CB, NW, L), jnp.int32),
        pltpu.VMEM((2 * NB,), jnp.int32),
        pltpu.SemaphoreType.DMA,
    ],
)
def _p1b(counts, offsets, binfo, cbuf, obuf, bbuf, sem):
    w = lax.axis_index("c") * NS + lax.axis_index("s")

    @pl.when(w == 0)
    def _():
        lane = lax.iota(jnp.int32, L)

        def chunk_body(cb, carry):
            pltpu.sync_copy(counts.at[pl.ds(cb * _SCAN_CB, _SCAN_CB)], cbuf)

            def bucket_body(bl, carry):
                bglob = cb * _SCAN_CB + bl
                blv = jnp.full((L,), bl, jnp.int32)

                def vec_body(j, run):
                    jv = jnp.full((L,), j, jnp.int32)
                    v = plsc.load_gather(cbuf, [blv, jv, lane])
                    cs = plsc.cumsum(v)
                    off = (carry + run) + cs - v
                    plsc.store_scatter(obuf, [blv, jv, lane], off)
                    return run + cs[L - 1]
                run = lax.fori_loop(0, NW, vec_body, jnp.int32(0))
                bgv = jnp.full((L,), bglob, jnp.int32)
                m0 = lane == 0
                plsc.store_scatter(bbuf, [bgv], jnp.full((L,), carry, jnp.int32),
                                   mask=m0)
                plsc.store_scatter(bbuf, [bgv + NB], jnp.full((L,), run, jnp.int32),
                                   mask=m0)
                return carry + run
            carry = lax.fori_loop(0, _SCAN_CB, bucket_body, carry)
            pltpu.sync_copy(obuf, offsets.at[pl.ds(cb * _SCAN_CB, _SCAN_CB)])
            return carry
        lax.fori_loop(0, NB // _SCAN_CB, chunk_body, jnp.int32(0))
        pltpu.sync_copy(bbuf, binfo)


@functools.partial(
    pl.kernel,
    out_type=jax.ShapeDtypeStruct((NREC + CR, RECW), jnp.float32),
    mesh=_mesh,
    compiler_params=_sc_params,
    scratch_types=[
        pltpu.VMEM((CP,), jnp.float32),
        pltpu.VMEM((CP,), jnp.float32),
        pltpu.VMEM((CP,), jnp.float32),
        pltpu.VMEM((CP,), jnp.float32),
        pltpu.VMEM((NB, L), jnp.int32),
        pltpu.VMEM((2, ITERS * L, RECW), jnp.float32),
        pltpu.VMEM((2, ITERS * L // 128, 128), jnp.int32),
        pltpu.SemaphoreType.DMA,
        pltpu.SemaphoreType.DMA,
    ],
)
def _p1c(ra, dec, vel, flx, offsets, rec, rb, db, vb, fb, cur, stage, slots,
         sem_in, sem_out):
    w = lax.axis_index("c") * NS + lax.axis_index("s")
    lane = lax.iota(jnp.int32, L)
    pltpu.sync_copy(offsets.at[:, w, :], cur)
    ndma = ITERS * L // 128

    def chunk_body(ch, _):
        base = w * P_TILE + ch * CP
        cps = [pltpu.async_copy(src.at[pl.ds(base, CP)], dst, sem_in)
               for src, dst in ((ra, rb), (dec, db), (vel, vb), (flx, fb))]
        for cp in cps:
            cp.wait()
        pg = ch & 1

        def it_body(i, _):
            sl = pl.ds(i * L, L)
            b, idx, fx, fy, fv, f = _point_math(rb[sl], db[sl], vb[sl], fb[sl])
            slot = plsc.load_gather(cur, [b, lane])
            plsc.store_scatter(cur, [b, lane], slot + 1)
            row = jnp.full((L,), i * L, jnp.int32) + lane
            idxf = plsc.bitcast(idx, jnp.float32)
            pgv = jnp.full((L,), pg, jnp.int32)
            for col, valv in ((0, idxf), (1, fx), (2, fy),
                              (3, f * (1.0 - fv)), (4, f * fv)):
                plsc.store_scatter(stage, [pgv, row, jnp.full((L,), col, jnp.int32)],
                                   valv)
            plsc.store_scatter(slots,
                               [pgv, jnp.full((L,), i >> 3, jnp.int32),
                                jnp.full((L,), (i & 7) * L, jnp.int32) + lane],
                               slot)
            return 0
        lax.fori_loop(0, ITERS, it_body, 0)

        def fire(j, _):
            pltpu.async_copy(stage.at[pg, pl.ds(j * 128, 128)],
                             rec.at[slots.at[pg, j]], sem_out)
            return 0
        lax.fori_loop(0, ndma, fire, 0)

        @pl.when(ch >= 1)
        def _():
            og = 1 - pg
            pltpu.make_async_copy(rec.at[pl.ds(0, ITERS * L)], stage.at[og],
                                  sem_out).wait()
        return 0
    lax.fori_loop(0, NCHUNK, chunk_body, 0)
    pltpu.make_async_copy(rec.at[pl.ds(0, ITERS * L)],
                          stage.at[(NCHUNK - 1) & 1], sem_out).wait()


@functools.partial(
    pl.kernel,
    out_type=(
        jax.ShapeDtypeStruct((NV * NPIX * NPIX,), jnp.float32),
        jax.ShapeDtypeStruct((NB, NPIX), jnp.float32),
    ),
    mesh=_mesh,
    compiler_params=_sc_params,
    scratch_types=[
        pltpu.VMEM((ACC,), jnp.float32),
        pltpu.VMEM((2, CR, RECW), jnp.float32),
        pltpu.VMEM((2 * NB,), jnp.int32),
        pltpu.SemaphoreType.DMA,
    ],
)
def _p2(rec, binfo, cube, halos, acc, buf, bi, sem):
    w = lax.axis_index("c") * NS + lax.axis_index("s")
    lane = lax.iota(jnp.int32, L)
    pltpu.sync_copy(binfo, bi)
    zeros = jnp.zeros((L,), jnp.float32)

    def block_body(blk, _):
        bid = w * 8 + blk
        c = bid >> 2
        q = bid & 3

        def zb(i, _):
            for u in range(4):
                acc[pl.ds(i * (4 * L) + u * L, L)] = zeros
            return 0
        lax.fori_loop(0, ACC // (4 * L), zb, 0)

        def do_half(bucket, valcol, enable):
            bkc = jnp.clip(bucket, 0, NB - 1)
            start = plsc.load_gather(bi, [jnp.full((L,), bkc, jnp.int32)])[0]
            n = plsc.load_gather(bi, [jnp.full((L,), bkc + NB, jnp.int32)])[0]
            n = jnp.where(enable, n, 0)
            nch = (n + (CR - 1)) >> 10

            @pl.when(nch > 0)
            def _():
                pltpu.async_copy(rec.at[pl.ds(start, CR)], buf.at[0], sem)

            def ch_body(ch, _):
                pg = ch & 1
                pltpu.make_async_copy(rec.at[pl.ds(start + ch * CR, CR)],
                                      buf.at[pg], sem).wait()

                @pl.when(ch + 1 < nch)
                def _():
                    pltpu.async_copy(rec.at[pl.ds(start + (ch + 1) * CR, CR)],
                                     buf.at[1 - pg], sem)
                rem0 = n - ch * CR

                def grp(g, _):
                    row = jnp.full((L,), g * L, jnp.int32) + lane
                    gv = lambda col: plsc.load_gather(
                        buf, [jnp.full((L,), pg, jnp.int32), row,
                              jnp.full((L,), col, jnp.int32)])
                    idx = plsc.bitcast(gv(0), jnp.int32)
                    fx = gv(1)
                    fy = gv(2)
                    val = gv(valcol)
                    m = row < rem0
                    wy0 = val * (1.0 - fy)
                    wy1 = val * fy
                    plsc.addupdate_scatter(acc, [idx], wy0 * (1.0 - fx), mask=m)
                    plsc.addupdate_scatter(acc, [idx + 1], wy0 * fx, mask=m)
                    plsc.addupdate_scatter(acc, [idx + NPIX], wy1 * (1.0 - fx),
                                           mask=m)
                    plsc.addupdate_scatter(acc, [idx + NPIX + 1], wy1 * fx,
                                           mask=m)
                    return 0
                lax.fori_loop(0, CR // L, grp, 0)
                return 0
            lax.fori_loop(0, nch, ch_body, 0)

        do_half(c * 4 + q, 3, c <= NV - 2)
        do_half((c - 1) * 4 + q, 4, c >= 1)
        pltpu.sync_copy(acc.at[pl.ds(0, 128 * NPIX)],
                        cube.at[pl.ds((c * NPIX + q * 128) * NPIX, 128 * NPIX)])
        pltpu.sync_copy(acc.at[pl.ds(128 * NPIX, NPIX)], halos.at[bid])
        return 0
    lax.fori_loop(0, 8, block_body, 0)


def _p3_body(c_ref, h_ref, o_ref):
    o_ref[...] = c_ref[...]
    o_ref[0, 128, :] = c_ref[0, 128, :] + h_ref[0, 0, :]
    o_ref[0, 256, :] = c_ref[0, 256, :] + h_ref[0, 1, :]
    o_ref[0, 384, :] = c_ref[0, 384, :] + h_ref[0, 2, :]


def _p3(cube_raw, halos):
    return pl.pallas_call(
        _p3_body,
        grid=(NV,),
        in_specs=[
            pl.BlockSpec((1, NPIX, NPIX), lambda i: (i, 0, 0)),
            pl.BlockSpec((1, 4, NPIX), lambda i: (i, 0, 0)),
        ],
        out_specs=pl.BlockSpec((1, NPIX, NPIX), lambda i: (i, 0, 0)),
        out_shape=jax.ShapeDtypeStruct((NV, NPIX, NPIX), jnp.float32),
    )(cube_raw, halos)


def kernel(pos_img, vel_chan, flux):
    ra = pos_img[..., 0].reshape(M)
    dec = pos_img[..., 1].reshape(M)
    v = vel_chan.reshape(M)
    f = flux.reshape(M)
    pad = M_PAD - M
    ra = jnp.concatenate([ra, jnp.zeros((pad,), jnp.float32)])
    dec = jnp.concatenate([dec, jnp.zeros((pad,), jnp.float32)])
    v = jnp.concatenate([v, jnp.zeros((pad,), jnp.float32)])
    f = jnp.concatenate([f, jnp.zeros((pad,), jnp.float32)])

    counts = _p1a(ra, dec, v, f)
    offsets, binfo = _p1b(counts)
    rec = _p1c(ra, dec, v, f, offsets)
    cube_raw, halos = _p2(rec, binfo)
    cube = _p3(cube_raw.reshape(NV, NPIX, NPIX), halos.reshape(NV, 4, NPIX))
    return cube


# speedup vs baseline: 1.2445x; 1.0435x over previous
import functools

import jax
import jax.numpy as jnp
from jax import lax
from jax.experimental import pallas as pl
from jax.experimental.pallas import tpu as pltpu
from jax.experimental.pallas import tpu_sc as plsc

NV = 64
NPIX = 512
PIXSCALE = 0.5
VEL0 = 0.0
DV = 10.0
FOV_HALF = 0.5 * (NPIX - 1) * PIXSCALE

NC, NS, L = 2, 16, 16
NW = NC * NS
M = 250000 * 8
CP = 2048
ITERS = CP // L
NCHUNK = 31
P_TILE = CP * NCHUNK
M_PAD = P_TILE * NW
NB = 256
NREC = M_PAD
RECW = 16
CR = 1024
ACC = 129 * 512

_mesh = plsc.VectorSubcoreMesh(core_axis_name="c", subcore_axis_name="s",
                               num_cores=NC, num_subcores=NS)
_sc_params = pltpu.CompilerParams(needs_layout_passes=False,
                                  use_tc_tiling_on_sc=False)


def _point_math(r, d, v, f):
    x = r * (1.0 / PIXSCALE) + (FOV_HALF / PIXSCALE)
    y = d * (1.0 / PIXSCALE) + (FOV_HALF / PIXSCALE)
    vz = v * (1.0 / DV) + (-VEL0 / DV)
    ok = ((x >= 0.0) & (x < NPIX - 1.0) & (y >= 0.0) & (y < NPIX - 1.0)
          & (vz >= 0.0) & (vz < NV - 1.0))
    f = jnp.where(ok, f, 0.0)
    ix = jnp.clip(x.astype(jnp.int32), 0, NPIX - 2)
    iy = jnp.clip(y.astype(jnp.int32), 0, NPIX - 2)
    iv = jnp.clip(vz.astype(jnp.int32), 0, NV - 2)
    fx = jnp.clip(x - ix.astype(jnp.float32), 0.0, 1.0)
    fy = jnp.clip(y - iy.astype(jnp.float32), 0.0, 1.0)
    fv = jnp.clip(vz - iv.astype(jnp.float32), 0.0, 1.0)
    yq = iy >> 7
    ly = iy & 127
    idx = ly * NPIX + ix
    b = iv * 4 + yq
    return b, idx, fx, fy, fv, f


@functools.partial(
    pl.kernel,
    out_type=jax.ShapeDtypeStruct((NB, NW, L), jnp.int32),
    mesh=_mesh,
    compiler_params=_sc_params,
    scratch_types=[
        pltpu.VMEM((2, CP), jnp.float32),
        pltpu.VMEM((2, CP), jnp.float32),
        pltpu.VMEM((2, CP), jnp.float32),
        pltpu.VMEM((2, CP), jnp.float32),
        pltpu.VMEM((NB, L), jnp.int32),
        pltpu.SemaphoreType.DMA,
    ],
)
def _p1a(ra, dec, vel, flx, counts, rb, db, vb, fb, cnt, sem):
    w = lax.axis_index("c") * NS + lax.axis_index("s")
    lane = lax.iota(jnp.int32, L)
    ones = jnp.ones((L,), jnp.int32)

    def zero_body(i, _):
        plsc.store_scatter(cnt, [jnp.full((L,), i, jnp.int32), lane],
                           jnp.zeros((L,), jnp.int32))
        return 0
    lax.fori_loop(0, NB, zero_body, 0)

    def in_copies(ch, pg):
        base = w * P_TILE + ch * CP
        return [pltpu.make_async_copy(src.at[pl.ds(base, CP)], dst.at[pg], sem)
                for src, dst in ((ra, rb), (dec, db), (vel, vb), (flx, fb))]

    for cp in in_copies(0, 0):
        cp.start()

    def chunk_body(ch, _):
        pg = ch & 1
        for cp in in_copies(ch, pg):
            cp.wait()

        @pl.when(ch + 1 < NCHUNK)
        def _():
            for cp in in_copies(ch + 1, 1 - pg):
                cp.start()

        def it_body(i, _):
            sl = pl.ds(i * L, L)
            b, _idx, _fx, _fy, _fv, _f = _point_math(
                rb[pg, sl], db[pg, sl], vb[pg, sl], fb[pg, sl])
            plsc.addupdate_scatter(cnt, [b, lane], ones)
            return 0
        lax.fori_loop(0, ITERS, it_body, 0)
        return 0
    lax.fori_loop(0, NCHUNK, chunk_body, 0)
    pltpu.sync_copy(cnt, counts.at[:, w, :])


_SCAN_CB = 16


@functools.partial(
    pl.kernel,
    out_type=(
        jax.ShapeDtypeStruct((NB, NW, L), jnp.int32),
        jax.ShapeDtypeStruct((2 * NB,), jnp.int32),
    ),
    mesh=_mesh,
    compiler_params=_sc_params,
    scratch_types=[
        pltpu.VMEM((_SCAN_CB, NW, L), jnp.int32),
        pltpu.VMEM((_SCAN_CB, NW, L), jnp.int32),
        pltpu.VMEM((2 * NB,), jnp.int32),
        pltpu.SemaphoreType.DMA,
    ],
)
def _p1b(counts, offsets, binfo, cbuf, obuf, bbuf, sem):
    w = lax.axis_index("c") * NS + lax.axis_index("s")

    @pl.when(w == 0)
    def _():
        lane = lax.iota(jnp.int32, L)

        def chunk_body(cb, carry):
            pltpu.sync_copy(counts.at[pl.ds(cb * _SCAN_CB, _SCAN_CB)], cbuf)

            def bucket_body(bl, carry):
                bglob = cb * _SCAN_CB + bl
                blv = jnp.full((L,), bl, jnp.int32)

                def vec_body(j, run):
                    jv = jnp.full((L,), j, jnp.int32)
                    v = plsc.load_gather(cbuf, [blv, jv, lane])
                    cs = plsc.cumsum(v)
                    off = (carry + run) + cs - v
                    plsc.store_scatter(obuf, [blv, jv, lane], off)
                    return run + cs[L - 1]
                run = lax.fori_loop(0, NW, vec_body, jnp.int32(0))
                bgv = jnp.full((L,), bglob, jnp.int32)
                m0 = lane == 0
                plsc.store_scatter(bbuf, [bgv], jnp.full((L,), carry, jnp.int32),
                                   mask=m0)
                plsc.store_scatter(bbuf, [bgv + NB], jnp.full((L,), run, jnp.int32),
                                   mask=m0)
                return carry + run
            carry = lax.fori_loop(0, _SCAN_CB, bucket_body, carry)
            pltpu.sync_copy(obuf, offsets.at[pl.ds(cb * _SCAN_CB, _SCAN_CB)])
            return carry
        lax.fori_loop(0, NB // _SCAN_CB, chunk_body, jnp.int32(0))
        pltpu.sync_copy(bbuf, binfo)


@functools.partial(
    pl.kernel,
    out_type=jax.ShapeDtypeStruct((NREC + CR, RECW), jnp.float32),
    mesh=_mesh,
    compiler_params=_sc_params,
    scratch_types=[
        pltpu.VMEM((2, CP), jnp.float32),
        pltpu.VMEM((2, CP), jnp.float32),
        pltpu.VMEM((2, CP), jnp.float32),
        pltpu.VMEM((2, CP), jnp.float32),
        pltpu.VMEM((NB, L), jnp.int32),
        pltpu.VMEM((2, ITERS * L, RECW), jnp.float32),
        pltpu.VMEM((2, ITERS * L // 128, 128), jnp.int32),
        pltpu.SemaphoreType.DMA,
        pltpu.SemaphoreType.DMA,
    ],
)
def _p1c(ra, dec, vel, flx, offsets, rec, rb, db, vb, fb, cur, stage, slots,
         sem_in, sem_out):
    w = lax.axis_index("c") * NS + lax.axis_index("s")
    lane = lax.iota(jnp.int32, L)
    pltpu.sync_copy(offsets.at[:, w, :], cur)
    ndma = ITERS * L // 128

    def in_copies(ch, pg):
        base = w * P_TILE + ch * CP
        return [pltpu.make_async_copy(src.at[pl.ds(base, CP)], dst.at[pg],
                                      sem_in)
                for src, dst in ((ra, rb), (dec, db), (vel, vb), (flx, fb))]

    for cp in in_copies(0, 0):
        cp.start()

    def chunk_body(ch, _):
        pg = ch & 1
        for cp in in_copies(ch, pg):
            cp.wait()

        @pl.when(ch + 1 < NCHUNK)
        def _():
            for cp in in_copies(ch + 1, 1 - pg):
                cp.start()

        def it_body(i, _):
            sl = pl.ds(i * L, L)
            b, idx, fx, fy, fv, f = _point_math(
                rb[pg, sl], db[pg, sl], vb[pg, sl], fb[pg, sl])
            slot = plsc.load_gather(cur, [b, lane])
            plsc.store_scatter(cur, [b, lane], slot + 1)
            row = jnp.full((L,), i * L, jnp.int32) + lane
            idxf = plsc.bitcast(idx, jnp.float32)
            pgv = jnp.full((L,), pg, jnp.int32)
            for col, valv in ((0, idxf), (1, fx), (2, fy),
                              (3, f * (1.0 - fv)), (4, f * fv)):
                plsc.store_scatter(stage, [pgv, row, jnp.full((L,), col, jnp.int32)],
                                   valv)
            plsc.store_scatter(slots,
                               [pgv, jnp.full((L,), i >> 3, jnp.int32),
                                jnp.full((L,), (i & 7) * L, jnp.int32) + lane],
                               slot)
            return 0
        lax.fori_loop(0, ITERS, it_body, 0)

        def fire(j, _):
            pltpu.async_copy(stage.at[pg, pl.ds(j * 128, 128)],
                             rec.at[slots.at[pg, j]], sem_out)
            return 0
        lax.fori_loop(0, ndma, fire, 0)

        @pl.when(ch >= 1)
        def _():
            og = 1 - pg
            pltpu.make_async_copy(rec.at[pl.ds(0, ITERS * L)], stage.at[og],
                                  sem_out).wait()
        return 0
    lax.fori_loop(0, NCHUNK, chunk_body, 0)
    pltpu.make_async_copy(rec.at[pl.ds(0, ITERS * L)],
                          stage.at[(NCHUNK - 1) & 1], sem_out).wait()


@functools.partial(
    pl.kernel,
    out_type=(
        jax.ShapeDtypeStruct((NV * NPIX * NPIX,), jnp.float32),
        jax.ShapeDtypeStruct((NB, NPIX), jnp.float32),
    ),
    mesh=_mesh,
    compiler_params=_sc_params,
    scratch_types=[
        pltpu.VMEM((ACC,), jnp.float32),
        pltpu.VMEM((2, CR, RECW), jnp.float32),
        pltpu.VMEM((2 * NB,), jnp.int32),
        pltpu.SemaphoreType.DMA,
    ],
)
def _p2(rec, binfo, cube, halos, acc, buf, bi, sem):
    w = lax.axis_index("c") * NS + lax.axis_index("s")
    lane = lax.iota(jnp.int32, L)
    pltpu.sync_copy(binfo, bi)
    zeros = jnp.zeros((L,), jnp.float32)

    def block_body(blk, _):
        bid = w * 8 + blk
        c = bid >> 2
        q = bid & 3

        def zb(i, _):
            for u in range(4):
                acc[pl.ds(i * (4 * L) + u * L, L)] = zeros
            return 0
        lax.fori_loop(0, ACC // (4 * L), zb, 0)

        def do_half(bucket, valcol, enable):
            bkc = jnp.clip(bucket, 0, NB - 1)
            start = plsc.load_gather(bi, [jnp.full((L,), bkc, jnp.int32)])[0]
            n = plsc.load_gather(bi, [jnp.full((L,), bkc + NB, jnp.int32)])[0]
            n = jnp.where(enable, n, 0)
            nch = (n + (CR - 1)) >> 10

            @pl.when(nch > 0)
            def _():
                pltpu.async_copy(rec.at[pl.ds(start, CR)], buf.at[0], sem)

            def ch_body(ch, _):
                pg = ch & 1
                pltpu.make_async_copy(rec.at[pl.ds(start + ch * CR, CR)],
                                      buf.at[pg], sem).wait()

                @pl.when(ch + 1 < nch)
                def _():
                    pltpu.async_copy(rec.at[pl.ds(start + (ch + 1) * CR, CR)],
                                     buf.at[1 - pg], sem)
                rem0 = n - ch * CR

                def grp(g, _):
                    row = jnp.full((L,), g * L, jnp.int32) + lane
                    gv = lambda col: plsc.load_gather(
                        buf, [jnp.full((L,), pg, jnp.int32), row,
                              jnp.full((L,), col, jnp.int32)])
                    idx = plsc.bitcast(gv(0), jnp.int32)
                    fx = gv(1)
                    fy = gv(2)
                    val = gv(valcol)
                    m = row < rem0
                    wy0 = val * (1.0 - fy)
                    wy1 = val * fy
                    plsc.addupdate_scatter(acc, [idx], wy0 * (1.0 - fx), mask=m)
                    plsc.addupdate_scatter(acc, [idx + 1], wy0 * fx, mask=m)
                    plsc.addupdate_scatter(acc, [idx + NPIX], wy1 * (1.0 - fx),
                                           mask=m)
                    plsc.addupdate_scatter(acc, [idx + NPIX + 1], wy1 * fx,
                                           mask=m)
                    return 0
                lax.fori_loop(0, CR // L, grp, 0)
                return 0
            lax.fori_loop(0, nch, ch_body, 0)

        do_half(c * 4 + q, 3, c <= NV - 2)
        do_half((c - 1) * 4 + q, 4, c >= 1)
        pltpu.sync_copy(acc.at[pl.ds(0, 128 * NPIX)],
                        cube.at[pl.ds((c * NPIX + q * 128) * NPIX, 128 * NPIX)])
        pltpu.sync_copy(acc.at[pl.ds(128 * NPIX, NPIX)], halos.at[bid])
        return 0
    lax.fori_loop(0, 8, block_body, 0)


def _p3_body(c_ref, h_ref, o_ref):
    o_ref[...] = c_ref[...]
    o_ref[0, 128, :] = c_ref[0, 128, :] + h_ref[0, 0, :]
    o_ref[0, 256, :] = c_ref[0, 256, :] + h_ref[0, 1, :]
    o_ref[0, 384, :] = c_ref[0, 384, :] + h_ref[0, 2, :]


def _p3(cube_raw, halos):
    return pl.pallas_call(
        _p3_body,
        grid=(NV,),
        in_specs=[
            pl.BlockSpec((1, NPIX, NPIX), lambda i: (i, 0, 0)),
            pl.BlockSpec((1, 4, NPIX), lambda i: (i, 0, 0)),
        ],
        out_specs=pl.BlockSpec((1, NPIX, NPIX), lambda i: (i, 0, 0)),
        out_shape=jax.ShapeDtypeStruct((NV, NPIX, NPIX), jnp.float32),
    )(cube_raw, halos)


def kernel(pos_img, vel_chan, flux):
    ra = pos_img[..., 0].reshape(M)
    dec = pos_img[..., 1].reshape(M)
    v = vel_chan.reshape(M)
    f = flux.reshape(M)
    pad = M_PAD - M
    ra = jnp.concatenate([ra, jnp.zeros((pad,), jnp.float32)])
    dec = jnp.concatenate([dec, jnp.zeros((pad,), jnp.float32)])
    v = jnp.concatenate([v, jnp.zeros((pad,), jnp.float32)])
    f = jnp.concatenate([f, jnp.zeros((pad,), jnp.float32)])

    counts = _p1a(ra, dec, v, f)
    offsets, binfo = _p1b(counts)
    rec = _p1c(ra, dec, v, f, offsets)
    cube_raw, halos = _p2(rec, binfo)
    cube = _p3(cube_raw.reshape(NV, NPIX, NPIX), halos.reshape(NV, 4, NPIX))
    return cube
